# Initial kernel scaffold; baseline (speedup 1.0000x reference)
import jax
import jax.numpy as jnp
from jax.experimental import pallas as pl


def kernel(prompt_tokens, anchors, counts):
    raise NotImplementedError("write your pallas kernel here")


# speedup vs baseline: 25.8945x; 25.8945x over previous
import functools

import jax
import jax.numpy as jnp
from jax.experimental import pallas as pl
from jax.experimental.pallas import tpu as pltpu

EMA_MOMENTUM = 0.9
NUM_ANCHORS = 8192
DESC_DIM = 256
B, N = 512, 256

BB = 16
KB = 1024
EPS = 1e-12


def _desc_body(pt_ref, d1_ref, x_ref):
    pt = pt_ref[...]
    s = pt.mean(axis=1)
    n1 = jnp.sqrt(jnp.sum(s * s, axis=-1, keepdims=True))
    d1 = s / jnp.maximum(n1, EPS)
    n2 = jnp.sqrt(jnp.sum(d1 * d1, axis=-1, keepdims=True))
    x = d1 / jnp.maximum(n2, EPS)
    d1_ref[...] = d1
    x_ref[...] = x


def _argmin_body(x_ref, anch_ref, ids_ref, best_ref):
    k = pl.program_id(0)
    x = x_ref[...]
    a = anch_ref[...]
    an = jnp.sqrt(jnp.sum(a * a, axis=-1, keepdims=True))
    a = a / jnp.maximum(an, EPS)
    scores = jax.lax.dot_general(
        x, a, (((1,), (1,)), ((), ())), preferred_element_type=jnp.float32)
    dist = 1.0 - scores
    lmin = jnp.min(dist, axis=-1, keepdims=True)
    col = jax.lax.broadcasted_iota(jnp.int32, dist.shape, 1)
    lidx = jnp.min(jnp.where(dist == lmin, col, NUM_ANCHORS),
                   axis=-1, keepdims=True) + k * KB

    @pl.when(k == 0)
    def _():
        best_ref[...] = lmin
        ids_ref[...] = lidx

    @pl.when(k > 0)
    def _():
        prev = best_ref[...]
        upd = lmin < prev
        best_ref[...] = jnp.where(upd, lmin, prev)
        ids_ref[...] = jnp.where(upd, lidx, ids_ref[...])


def _ema_body(ids_ref, desc_ref, anch_hbm_ref, cnt_in_ref, anch_ref, cnt_ref,
              sem):
    cp = pltpu.make_async_copy(anch_hbm_ref, anch_ref, sem)
    cp.start()
    cp.wait()
    cnt_ref[...] = cnt_in_ref[...]
    def body(b, _):
        kk = ids_ref[b, 0]
        c = cnt_ref[pl.ds(kk, 1), :]
        m = jnp.where(c < 1.0, 0.0, EMA_MOMENTUM)
        d = desc_ref[pl.ds(b, 1), :]
        arow = anch_ref[pl.ds(kk, 1), :]
        new = m * arow + (1.0 - m) * d
        nn = jnp.sqrt(jnp.sum(new * new))
        new = new / jnp.maximum(nn, EPS)
        anch_ref[pl.ds(kk, 1), :] = new
        cnt_ref[pl.ds(kk, 1), :] = c + 1.0
        return 0

    jax.lax.fori_loop(0, B, body, 0)


@jax.jit
def kernel(prompt_tokens, anchors, counts):
    d1, x = pl.pallas_call(
        _desc_body,
        grid=(B // BB,),
        in_specs=[pl.BlockSpec((BB, N, DESC_DIM), lambda i: (i, 0, 0))],
        out_specs=[pl.BlockSpec((BB, DESC_DIM), lambda i: (i, 0)),
                   pl.BlockSpec((BB, DESC_DIM), lambda i: (i, 0))],
        out_shape=[jax.ShapeDtypeStruct((B, DESC_DIM), jnp.float32),
                   jax.ShapeDtypeStruct((B, DESC_DIM), jnp.float32)],
    )(prompt_tokens)

    ids2d, _ = pl.pallas_call(
        _argmin_body,
        grid=(NUM_ANCHORS // KB,),
        in_specs=[pl.BlockSpec((B, DESC_DIM), lambda k: (0, 0)),
                  pl.BlockSpec((KB, DESC_DIM), lambda k: (k, 0))],
        out_specs=[pl.BlockSpec((B, 1), lambda k: (0, 0)),
                   pl.BlockSpec((B, 1), lambda k: (0, 0))],
        out_shape=[jax.ShapeDtypeStruct((B, 1), jnp.int32),
                   jax.ShapeDtypeStruct((B, 1), jnp.float32)],
    )(x, anchors)

    new_anchors, new_counts = pl.pallas_call(
        _ema_body,
        grid=(1,),
        in_specs=[pl.BlockSpec(memory_space=pltpu.SMEM),
                  pl.BlockSpec((B, DESC_DIM), lambda i: (0, 0)),
                  pl.BlockSpec(memory_space=pl.ANY),
                  pl.BlockSpec((NUM_ANCHORS, 1), lambda i: (0, 0))],
        out_specs=[pl.BlockSpec((NUM_ANCHORS, DESC_DIM), lambda i: (0, 0)),
                   pl.BlockSpec((NUM_ANCHORS, 1), lambda i: (0, 0))],
        out_shape=[jax.ShapeDtypeStruct((NUM_ANCHORS, DESC_DIM), jnp.float32),
                   jax.ShapeDtypeStruct((NUM_ANCHORS, 1), jnp.float32)],
        scratch_shapes=[pltpu.SemaphoreType.DMA],
    )(ids2d, d1, anchors, counts.reshape(NUM_ANCHORS, 1))

    return ids2d.reshape(B), new_anchors, new_counts.reshape(NUM_ANCHORS)
